# Initial kernel scaffold; baseline (speedup 1.0000x reference)
import jax
import jax.numpy as jnp
from jax.experimental import pallas as pl


def kernel(initial_points, enc_W, enc_b, layers, dec_W, dec_b):
    raise NotImplementedError("write your pallas kernel here")


# speedup vs baseline: 16.7845x; 16.7845x over previous
import functools

import jax
import jax.numpy as jnp
from jax.experimental import pallas as pl

DIM = 4
NHID = 64
N = 2048
RADIUS2 = 0.2 * 0.2
EPS = 1e-5
BD = 128
BS = 128

_INTERPRET = False


def _enc_kernel(x_ref, w_ref, b_ref, out_ref):
    out_ref[...] = (
        jnp.dot(x_ref[...], w_ref[...], preferred_element_type=jnp.float32)
        + b_ref[...]
    )


def _msg_kernel(xd_ref, xs_ref, hd_ref, hs_ref, w1t_ref, w1b_ref, b1_ref,
                w2_ref, b2_ref, out_ref):
    j = pl.program_id(1)
    A = jnp.dot(hd_ref[...], w1t_ref[...], preferred_element_type=jnp.float32)
    B = jnp.dot(hs_ref[...], w1b_ref[...], preferred_element_type=jnp.float32)
    H = jax.nn.relu(A[:, None, :] + B[None, :, :] + b1_ref[...])
    M = jax.nn.relu(
        jax.lax.dot_general(
            H, w2_ref[...], (((2,), (0,)), ((), ())),
            preferred_element_type=jnp.float32)
        + b2_ref[...]
    )
    xd = xd_ref[...]
    xs = xs_ref[...]
    d2 = jnp.zeros((BD, BS, 1), jnp.float32)
    for k in range(DIM):
        dk = xd[:, None, k:k + 1] - xs[None, :, k:k + 1]
        d2 = d2 + dk * dk
    mf = (d2 <= RADIUS2).astype(jnp.float32)
    contrib = jnp.sum(M * mf, axis=1)

    @pl.when(j == 0)
    def _():
        out_ref[...] = contrib

    @pl.when(j != 0)
    def _():
        out_ref[...] = out_ref[...] + contrib


def _upd_kernel(h_ref, agg_ref, u1t_ref, u1b_ref, b1_ref, u2_ref, b2_ref,
                out_ref):
    h = h_ref[...]
    U = jax.nn.relu(
        jnp.dot(h, u1t_ref[...], preferred_element_type=jnp.float32)
        + jnp.dot(agg_ref[...], u1b_ref[...], preferred_element_type=jnp.float32)
        + b1_ref[...]
    )
    U2 = jax.nn.relu(
        jnp.dot(U, u2_ref[...], preferred_element_type=jnp.float32)
        + b2_ref[...]
    )
    mean = jnp.mean(U2, axis=0, keepdims=True)
    c = U2 - mean
    var = jnp.mean(c * c, axis=0, keepdims=True)
    out_ref[...] = c / jnp.sqrt(var + EPS) + h


def _dec_kernel(h_ref, w_ref, b_ref, out_ref):
    out_ref[...] = jax.nn.sigmoid(
        jnp.dot(h_ref[...], w_ref[...], preferred_element_type=jnp.float32)
        + b_ref[...]
    )


def _disc_kernel(xd_ref, xs_ref, sum1_ref, sum2_ref):
    i = pl.program_id(0)
    j = pl.program_id(1)
    xd = xd_ref[...]
    xsT = xs_ref[...].T
    p = jnp.ones((BD, BS), jnp.float32)
    for k in range(DIM):
        p = p * (1.0 - jnp.maximum(xd[:, k:k + 1], xsT[k:k + 1, :]))
    ts = jnp.sum(jnp.sum(p, axis=1, keepdims=True), axis=0, keepdims=True)

    q = jnp.ones((BD, 1), jnp.float32)
    for k in range(DIM):
        xk = xd[:, k:k + 1]
        q = q * (1.0 - xk * xk)
    s1 = jnp.sum(q, axis=0, keepdims=True)

    first = (i == 0) & (j == 0)

    @pl.when(first)
    def _():
        sum2_ref[...] = ts
        sum1_ref[...] = s1

    @pl.when(jnp.logical_not(first))
    def _():
        sum2_ref[...] = sum2_ref[...] + ts

    @pl.when(jnp.logical_not(first) & (j == 0))
    def _():
        sum1_ref[...] = sum1_ref[...] + s1


def _full(shape):
    return pl.BlockSpec(shape, lambda *_: tuple(0 for _ in shape))


def kernel(initial_points, enc_W, enc_b, layers, dec_W, dec_b):
    x = initial_points
    nd = N // BD
    ns = N // BS

    h = pl.pallas_call(
        _enc_kernel,
        out_shape=jax.ShapeDtypeStruct((N, NHID), jnp.float32),
        interpret=_INTERPRET,
    )(x, enc_W, enc_b.reshape(1, NHID))

    msg_call = pl.pallas_call(
        _msg_kernel,
        grid=(nd, ns),
        in_specs=[
            pl.BlockSpec((BD, DIM), lambda i, j: (i, 0)),
            pl.BlockSpec((BS, DIM), lambda i, j: (j, 0)),
            pl.BlockSpec((BD, NHID), lambda i, j: (i, 0)),
            pl.BlockSpec((BS, NHID), lambda i, j: (j, 0)),
            _full((NHID, NHID)),
            _full((NHID, NHID)),
            _full((1, NHID)),
            _full((NHID, NHID)),
            _full((1, NHID)),
        ],
        out_specs=pl.BlockSpec((BD, NHID), lambda i, j: (i, 0)),
        out_shape=jax.ShapeDtypeStruct((N, NHID), jnp.float32),
        interpret=_INTERPRET,
    )

    upd_call = pl.pallas_call(
        _upd_kernel,
        out_shape=jax.ShapeDtypeStruct((N, NHID), jnp.float32),
        interpret=_INTERPRET,
    )

    for p in layers:
        agg = msg_call(
            x, x, h, h,
            p["m1W"][:NHID], p["m1W"][NHID:], p["m1b"].reshape(1, NHID),
            p["m2W"], p["m2b"].reshape(1, NHID),
        )
        h = upd_call(
            h, agg,
            p["u1W"][:NHID], p["u1W"][NHID:], p["u1b"].reshape(1, NHID),
            p["u2W"], p["u2b"].reshape(1, NHID),
        )

    x_dec = pl.pallas_call(
        _dec_kernel,
        out_shape=jax.ShapeDtypeStruct((N, DIM), jnp.float32),
        interpret=_INTERPRET,
    )(h, dec_W, dec_b.reshape(1, DIM))

    sum1, sum2 = pl.pallas_call(
        _disc_kernel,
        grid=(nd, ns),
        in_specs=[
            pl.BlockSpec((BD, DIM), lambda i, j: (i, 0)),
            pl.BlockSpec((BS, DIM), lambda i, j: (j, 0)),
        ],
        out_specs=[
            pl.BlockSpec((1, 1), lambda i, j: (0, 0)),
            pl.BlockSpec((1, 1), lambda i, j: (0, 0)),
        ],
        out_shape=[
            jax.ShapeDtypeStruct((1, 1), jnp.float32),
            jax.ShapeDtypeStruct((1, 1), jnp.float32),
        ],
        interpret=_INTERPRET,
    )(x_dec, x_dec)

    term1 = 3.0 ** (-DIM)
    term2 = 2.0 / N * (2.0 ** (1 - DIM)) * sum1[0, 0]
    term3 = sum2[0, 0] / float(N) ** 2
    loss = jnp.sqrt(jnp.clip(term1 - term2 + term3, 1e-8))
    X = x_dec.reshape(1, N, DIM)
    return (loss, X)


# speedup vs baseline: 16.8088x; 1.0014x over previous
import functools

import jax
import jax.numpy as jnp
from jax.experimental import pallas as pl
from jax.experimental.pallas import tpu as pltpu

DIM = 4
NHID = 64
N = 2048
RADIUS2 = 0.2 * 0.2
EPS = 1e-5
BD = 128
BS = 128

_INTERPRET = False


def _enc_kernel(x_ref, w_ref, b_ref, out_ref):
    out_ref[...] = (
        jnp.dot(x_ref[...], w_ref[...], preferred_element_type=jnp.float32)
        + b_ref[...]
    )


def _msg_kernel(xd_ref, xs_ref, hd_ref, hs_ref, w1t_ref, w1b_ref, b1_ref,
                w2_ref, b2_ref, out_ref):
    j = pl.program_id(1)
    A = jnp.dot(hd_ref[...], w1t_ref[...], preferred_element_type=jnp.float32)
    B = jnp.dot(hs_ref[...], w1b_ref[...], preferred_element_type=jnp.float32)
    H = jax.nn.relu(A[:, None, :] + B[None, :, :] + b1_ref[...])
    Hf = H.reshape(BD * BS, NHID)
    M = jax.nn.relu(
        jnp.dot(Hf, w2_ref[...], preferred_element_type=jnp.float32)
        + b2_ref[...]
    ).reshape(BD, BS, NHID)
    xd = xd_ref[...]
    xs = xs_ref[...]
    d2 = jnp.zeros((BD, BS, 1), jnp.float32)
    for k in range(DIM):
        dk = xd[:, None, k:k + 1] - xs[None, :, k:k + 1]
        d2 = d2 + dk * dk
    mf = (d2 <= RADIUS2).astype(jnp.float32)
    contrib = jnp.sum(M * mf, axis=1)

    @pl.when(j == 0)
    def _():
        out_ref[...] = contrib

    @pl.when(j != 0)
    def _():
        out_ref[...] = out_ref[...] + contrib


def _upd_kernel(h_ref, agg_ref, u1t_ref, u1b_ref, b1_ref, u2_ref, b2_ref,
                out_ref):
    h = h_ref[...]
    U = jax.nn.relu(
        jnp.dot(h, u1t_ref[...], preferred_element_type=jnp.float32)
        + jnp.dot(agg_ref[...], u1b_ref[...], preferred_element_type=jnp.float32)
        + b1_ref[...]
    )
    U2 = jax.nn.relu(
        jnp.dot(U, u2_ref[...], preferred_element_type=jnp.float32)
        + b2_ref[...]
    )
    mean = jnp.mean(U2, axis=0, keepdims=True)
    c = U2 - mean
    var = jnp.mean(c * c, axis=0, keepdims=True)
    out_ref[...] = c / jnp.sqrt(var + EPS) + h


def _dec_kernel(h_ref, w_ref, b_ref, out_ref):
    out_ref[...] = jax.nn.sigmoid(
        jnp.dot(h_ref[...], w_ref[...], preferred_element_type=jnp.float32)
        + b_ref[...]
    )


def _disc_kernel(xd_ref, xs_ref, sum1_ref, sum2_ref):
    i = pl.program_id(0)
    j = pl.program_id(1)
    xd = xd_ref[...]
    xsT = xs_ref[...].T
    p = jnp.ones((BD, BS), jnp.float32)
    for k in range(DIM):
        p = p * (1.0 - jnp.maximum(xd[:, k:k + 1], xsT[k:k + 1, :]))
    ts = jnp.sum(jnp.sum(p, axis=1, keepdims=True), axis=0, keepdims=True)

    q = jnp.ones((BD, 1), jnp.float32)
    for k in range(DIM):
        xk = xd[:, k:k + 1]
        q = q * (1.0 - xk * xk)
    s1 = jnp.sum(q, axis=0, keepdims=True)

    first = (i == 0) & (j == 0)

    @pl.when(first)
    def _():
        sum2_ref[...] = ts
        sum1_ref[...] = s1

    @pl.when(jnp.logical_not(first))
    def _():
        sum2_ref[...] = sum2_ref[...] + ts

    @pl.when(jnp.logical_not(first) & (j == 0))
    def _():
        sum1_ref[...] = sum1_ref[...] + s1


def _full(shape):
    return pl.BlockSpec(shape, lambda *_: tuple(0 for _ in shape))


def kernel(initial_points, enc_W, enc_b, layers, dec_W, dec_b):
    x = initial_points
    nd = N // BD
    ns = N // BS

    h = pl.pallas_call(
        _enc_kernel,
        out_shape=jax.ShapeDtypeStruct((N, NHID), jnp.float32),
        interpret=_INTERPRET,
    )(x, enc_W, enc_b.reshape(1, NHID))

    msg_call = pl.pallas_call(
        _msg_kernel,
        grid=(nd, ns),
        in_specs=[
            pl.BlockSpec((BD, DIM), lambda i, j: (i, 0)),
            pl.BlockSpec((BS, DIM), lambda i, j: (j, 0)),
            pl.BlockSpec((BD, NHID), lambda i, j: (i, 0)),
            pl.BlockSpec((BS, NHID), lambda i, j: (j, 0)),
            _full((NHID, NHID)),
            _full((NHID, NHID)),
            _full((1, NHID)),
            _full((NHID, NHID)),
            _full((1, NHID)),
        ],
        out_specs=pl.BlockSpec((BD, NHID), lambda i, j: (i, 0)),
        out_shape=jax.ShapeDtypeStruct((N, NHID), jnp.float32),
        compiler_params=pltpu.CompilerParams(
            dimension_semantics=("parallel", "arbitrary")),
        interpret=_INTERPRET,
    )

    upd_call = pl.pallas_call(
        _upd_kernel,
        out_shape=jax.ShapeDtypeStruct((N, NHID), jnp.float32),
        interpret=_INTERPRET,
    )

    for p in layers:
        agg = msg_call(
            x, x, h, h,
            p["m1W"][:NHID], p["m1W"][NHID:], p["m1b"].reshape(1, NHID),
            p["m2W"], p["m2b"].reshape(1, NHID),
        )
        h = upd_call(
            h, agg,
            p["u1W"][:NHID], p["u1W"][NHID:], p["u1b"].reshape(1, NHID),
            p["u2W"], p["u2b"].reshape(1, NHID),
        )

    x_dec = pl.pallas_call(
        _dec_kernel,
        out_shape=jax.ShapeDtypeStruct((N, DIM), jnp.float32),
        interpret=_INTERPRET,
    )(h, dec_W, dec_b.reshape(1, DIM))

    sum1, sum2 = pl.pallas_call(
        _disc_kernel,
        grid=(nd, ns),
        in_specs=[
            pl.BlockSpec((BD, DIM), lambda i, j: (i, 0)),
            pl.BlockSpec((BS, DIM), lambda i, j: (j, 0)),
        ],
        out_specs=[
            pl.BlockSpec((1, 1), lambda i, j: (0, 0)),
            pl.BlockSpec((1, 1), lambda i, j: (0, 0)),
        ],
        out_shape=[
            jax.ShapeDtypeStruct((1, 1), jnp.float32),
            jax.ShapeDtypeStruct((1, 1), jnp.float32),
        ],
        interpret=_INTERPRET,
    )(x_dec, x_dec)

    term1 = 3.0 ** (-DIM)
    term2 = 2.0 / N * (2.0 ** (1 - DIM)) * sum1[0, 0]
    term3 = sum2[0, 0] / float(N) ** 2
    loss = jnp.sqrt(jnp.clip(term1 - term2 + term3, 1e-8))
    X = x_dec.reshape(1, N, DIM)
    return (loss, X)


# speedup vs baseline: 98.6103x; 5.8666x over previous
import jax
import jax.numpy as jnp
from jax.experimental import pallas as pl
from jax.experimental.pallas import tpu as pltpu

DIM = 4
NHID = 64
N = 2048
RADIUS2 = 0.2 * 0.2
EPS = 1e-5
BD = 128
BS = 128

def _enc_kernel(x_ref, w_ref, b_ref, out_ref):
    out_ref[...] = (
        jnp.dot(x_ref[...], w_ref[...], preferred_element_type=jnp.float32)
        + b_ref[...]
    )


def _msg_kernel(keep_ref, xd_ref, xs_ref, hd_ref, hs_ref, w1t_ref, w1b_ref,
                b1_ref, w2_ref, b2_ref, out_ref):
    i = pl.program_id(0)
    j = pl.program_id(1)

    @pl.when(j == 0)
    def _():
        out_ref[...] = jnp.zeros_like(out_ref)

    @pl.when(keep_ref[i, j] != 0)
    def _():
        A = jnp.dot(hd_ref[...], w1t_ref[...],
                    preferred_element_type=jnp.float32)
        B = jnp.dot(hs_ref[...], w1b_ref[...],
                    preferred_element_type=jnp.float32)
        A2 = jnp.concatenate([A, A], axis=1)
        B2 = jnp.concatenate([B[:BS], B[BS:]], axis=1)
        H = jax.nn.relu(A2[:, None, :] + B2[None, :, :] + b1_ref[...])
        Hf = H.reshape(BD * BS, 2 * NHID)
        M = jax.nn.relu(
            jnp.dot(Hf, w2_ref[...], preferred_element_type=jnp.float32)
            + b2_ref[...]
        ).reshape(BD, BS, 2 * NHID)
        xd = xd_ref[...]
        xsT = xs_ref[...].T
        d2 = jnp.zeros((BD, 2 * BS), jnp.float32)
        for k in range(DIM):
            dk = xd[:, k:k + 1] - xsT[k:k + 1, :]
            d2 = d2 + dk * dk
        mf = (d2 <= RADIUS2).astype(jnp.float32)
        sa = jnp.sum(M * mf[:, :BS, None], axis=1)
        sb = jnp.sum(M * mf[:, BS:, None], axis=1)
        out_ref[...] = out_ref[...] + (sa[:, :NHID] + sb[:, NHID:])


def _upd_kernel(h_ref, agg_ref, u1t_ref, u1b_ref, b1_ref, u2_ref, b2_ref,
                out_ref):
    h = h_ref[...]
    U = jax.nn.relu(
        jnp.dot(h, u1t_ref[...], preferred_element_type=jnp.float32)
        + jnp.dot(agg_ref[...], u1b_ref[...], preferred_element_type=jnp.float32)
        + b1_ref[...]
    )
    U2 = jax.nn.relu(
        jnp.dot(U, u2_ref[...], preferred_element_type=jnp.float32)
        + b2_ref[...]
    )
    mean = jnp.mean(U2, axis=0, keepdims=True)
    c = U2 - mean
    var = jnp.mean(c * c, axis=0, keepdims=True)
    out_ref[...] = c / jnp.sqrt(var + EPS) + h


def _dec_kernel(h_ref, w_ref, b_ref, out_ref):
    out_ref[...] = jax.nn.sigmoid(
        jnp.dot(h_ref[...], w_ref[...], preferred_element_type=jnp.float32)
        + b_ref[...]
    )


def _disc_kernel(xd_ref, xs_ref, sum1_ref, sum2_ref):
    i = pl.program_id(0)
    j = pl.program_id(1)
    xd = xd_ref[...]
    xsT = xs_ref[...].T
    p = jnp.ones((BD, BS), jnp.float32)
    for k in range(DIM):
        p = p * (1.0 - jnp.maximum(xd[:, k:k + 1], xsT[k:k + 1, :]))
    ts = jnp.sum(jnp.sum(p, axis=1, keepdims=True), axis=0, keepdims=True)

    q = jnp.ones((BD, 1), jnp.float32)
    for k in range(DIM):
        xk = xd[:, k:k + 1]
        q = q * (1.0 - xk * xk)
    s1 = jnp.sum(q, axis=0, keepdims=True)

    first = (i == 0) & (j == 0)

    @pl.when(first)
    def _():
        sum2_ref[...] = ts
        sum1_ref[...] = s1

    @pl.when(jnp.logical_not(first))
    def _():
        sum2_ref[...] = sum2_ref[...] + ts

    @pl.when(jnp.logical_not(first) & (j == 0))
    def _():
        sum1_ref[...] = sum1_ref[...] + s1


def _full(shape):
    return pl.BlockSpec(shape, lambda *_: tuple(0 for _ in shape))


def kernel(initial_points, enc_W, enc_b, layers, dec_W, dec_b):
    perm = jnp.argsort(initial_points[:, 0])
    x = initial_points[perm]
    nd = N // BD
    ns2 = N // (2 * BS)

    x0 = x[:, 0]
    dblk = x0.reshape(nd, BD)
    sblk = x0.reshape(ns2, 2 * BS)
    dmin, dmax = dblk[:, 0], dblk[:, -1]
    smin, smax = sblk[:, 0], sblk[:, -1]
    gap = jnp.maximum(
        jnp.maximum(smin[None, :] - dmax[:, None], dmin[:, None] - smax[None, :]),
        0.0)
    keep = (gap * gap <= RADIUS2).astype(jnp.int32)

    h = pl.pallas_call(
        _enc_kernel,
        out_shape=jax.ShapeDtypeStruct((N, NHID), jnp.float32),
    )(x, enc_W, enc_b.reshape(1, NHID))

    msg_call = pl.pallas_call(
        _msg_kernel,
        grid_spec=pltpu.PrefetchScalarGridSpec(
            num_scalar_prefetch=1,
            grid=(nd, ns2),
            in_specs=[
                pl.BlockSpec((BD, DIM), lambda i, j, k: (i, 0)),
                pl.BlockSpec((2 * BS, DIM), lambda i, j, k: (j, 0)),
                pl.BlockSpec((BD, NHID), lambda i, j, k: (i, 0)),
                pl.BlockSpec((2 * BS, NHID), lambda i, j, k: (j, 0)),
                _full((NHID, NHID)),
                _full((NHID, NHID)),
                _full((1, 2 * NHID)),
                _full((2 * NHID, 2 * NHID)),
                _full((1, 2 * NHID)),
            ],
            out_specs=pl.BlockSpec((BD, NHID), lambda i, j, k: (i, 0)),
        ),
        out_shape=jax.ShapeDtypeStruct((N, NHID), jnp.float32),
        compiler_params=pltpu.CompilerParams(
            dimension_semantics=("parallel", "arbitrary")),
    )

    upd_call = pl.pallas_call(
        _upd_kernel,
        out_shape=jax.ShapeDtypeStruct((N, NHID), jnp.float32),
    )

    zero = jnp.zeros((NHID, NHID), jnp.float32)
    for p in layers:
        w2d = jnp.block([[p["m2W"], zero], [zero, p["m2W"]]])
        b1d = jnp.tile(p["m1b"], 2).reshape(1, 2 * NHID)
        b2d = jnp.tile(p["m2b"], 2).reshape(1, 2 * NHID)
        agg = msg_call(
            keep, x, x, h, h,
            p["m1W"][:NHID], p["m1W"][NHID:], b1d, w2d, b2d,
        )
        h = upd_call(
            h, agg,
            p["u1W"][:NHID], p["u1W"][NHID:], p["u1b"].reshape(1, NHID),
            p["u2W"], p["u2b"].reshape(1, NHID),
        )

    x_dec = pl.pallas_call(
        _dec_kernel,
        out_shape=jax.ShapeDtypeStruct((N, DIM), jnp.float32),
    )(h, dec_W, dec_b.reshape(1, DIM))

    sum1, sum2 = pl.pallas_call(
        _disc_kernel,
        grid=(nd, N // BS),
        in_specs=[
            pl.BlockSpec((BD, DIM), lambda i, j: (i, 0)),
            pl.BlockSpec((BS, DIM), lambda i, j: (j, 0)),
        ],
        out_specs=[
            pl.BlockSpec((1, 1), lambda i, j: (0, 0)),
            pl.BlockSpec((1, 1), lambda i, j: (0, 0)),
        ],
        out_shape=[
            jax.ShapeDtypeStruct((1, 1), jnp.float32),
            jax.ShapeDtypeStruct((1, 1), jnp.float32),
        ],
    )(x_dec, x_dec)

    term1 = 3.0 ** (-DIM)
    term2 = 2.0 / N * (2.0 ** (1 - DIM)) * sum1[0, 0]
    term3 = sum2[0, 0] / float(N) ** 2
    loss = jnp.sqrt(jnp.clip(term1 - term2 + term3, 1e-8))
    inv = jnp.argsort(perm)
    X = x_dec[inv].reshape(1, N, DIM)
    return (loss, X)
